# baseline (speedup 1.0000x reference)
import functools

import jax
import jax.numpy as jnp
from jax import lax
from jax.experimental import pallas as pl
from jax.experimental.pallas import tpu as pltpu
from jax.experimental.pallas import tpu_sc as plsc

COILS = 8
NZ, NY, NX = 32, 256, 256
NTOT = NZ * NY * NX
NC, NS = 2, 16
NW = NC * NS
ROWS = 64
CH = ROWS * NX
NCHUNK = (NY * NX) // CH


def _sc_gather(x2, idx_z, idx_y, idx_x):
    mesh = plsc.VectorSubcoreMesh(core_axis_name="c", subcore_axis_name="s")

    @functools.partial(
        pl.kernel,
        out_type=jax.ShapeDtypeStruct((COILS, NZ, NY, NX), jnp.float32),
        mesh=mesh,
        compiler_params=pltpu.CompilerParams(use_tc_tiling_on_sc=False),
        scratch_types=[
            pltpu.VMEM((ROWS, NX), jnp.int32),
            pltpu.VMEM((ROWS, NX), jnp.int32),
            pltpu.VMEM((ROWS, NX), jnp.int32),
            pltpu.VMEM((ROWS, NX), jnp.int32),
            pltpu.VMEM((ROWS, NX), jnp.float32),
            pltpu.SemaphoreType.DMA,
        ],
    )
    def k(x_hbm, iz_hbm, iy_hbm, ix_hbm, out_hbm,
          iz_v, iy_v, ix_v, flat2, rows2, sem):
        wid = lax.axis_index("s") * NC + lax.axis_index("c")
        for chunk in range(NCHUNK):
            r0 = chunk * ROWS
            pltpu.sync_copy(iz_hbm.at[wid, pl.ds(r0, ROWS), :], iz_v)
            pltpu.sync_copy(iy_hbm.at[wid, pl.ds(r0, ROWS), :], iy_v)
            pltpu.sync_copy(ix_hbm.at[wid, pl.ds(r0, ROWS), :], ix_v)

            def body(j, _):
                r = j >> 4
                c0 = (j & 15) * 16
                s = pl.ds(c0, 16)
                flat2[r, s] = (iz_v[r, s] << 16) | (iy_v[r, s] << 8) | ix_v[r, s]
                return 0

            lax.fori_loop(0, CH // 16, body, 0)

            for c in range(COILS):
                def fire(r, _):
                    pltpu.async_copy(
                        x_hbm.at[c].at[flat2.at[r]], rows2.at[r], sem
                    )
                    return 0

                lax.fori_loop(0, ROWS, fire, 0)

                def drain(r, _):
                    pltpu.make_async_copy(
                        x_hbm.at[c].at[flat2.at[r]], rows2.at[r], sem
                    ).wait()
                    return 0

                lax.fori_loop(0, ROWS, drain, 0)
                pltpu.sync_copy(rows2, out_hbm.at[c, wid, pl.ds(r0, ROWS), :])

    return k(x2, idx_z, idx_y, idx_x)


def kernel(x, idx_z, idx_y, idx_x):
    return (_sc_gather(x.reshape(COILS, NTOT), idx_z, idx_y, idx_x),)


# speedup vs baseline: 1.3388x; 1.3388x over previous
import functools

import jax
import jax.numpy as jnp
from jax import lax
from jax.experimental import pallas as pl
from jax.experimental.pallas import tpu as pltpu
from jax.experimental.pallas import tpu_sc as plsc

COILS = 8
NZ, NY, NX = 32, 256, 256
NTOT = NZ * NY * NX
NC, NS = 2, 16
NW = NC * NS

TV = 4096
TCHUNK = NTOT // NW // TV

ROWS = 32
GR = 16
NCHUNK = NY // ROWS

_MESH = dict(core_axis_name="c", subcore_axis_name="s")


def _transpose(x2):
    @functools.partial(
        pl.kernel,
        out_type=jax.ShapeDtypeStruct((NTOT, COILS), jnp.float32),
        mesh=plsc.VectorSubcoreMesh(**_MESH),
        compiler_params=pltpu.CompilerParams(
            use_tc_tiling_on_sc=False, needs_layout_passes=False
        ),
        scratch_types=[
            pltpu.VMEM((COILS, TV), jnp.float32),
            pltpu.VMEM((TV, COILS), jnp.float32),
            pltpu.SemaphoreType.DMA,
        ],
    )
    def k(x_hbm, xt_hbm, xin, xout, sem):
        wid = lax.axis_index("s") * NC + lax.axis_index("c")
        lanes = lax.iota(jnp.int32, 16)
        for chunk in range(TCHUNK):
            vbase = (wid * TCHUNK + chunk) * TV
            for c in range(COILS):
                pltpu.async_copy(
                    x_hbm.at[c, pl.ds(vbase, TV)], xin.at[c], sem
                )
            for c in range(COILS):
                pltpu.make_async_copy(
                    x_hbm.at[c, pl.ds(vbase, TV)], xin.at[c], sem
                ).wait()

            for c in range(COILS):
                cvec = jnp.full((16,), c, jnp.int32)

                def body(i, _, c=c, cvec=cvec):
                    v0 = i * 16
                    val = xin[c, pl.ds(v0, 16)]
                    plsc.store_scatter(xout, [v0 + lanes, cvec], val)
                    return 0

                lax.fori_loop(0, TV // 16, body, 0)

            pltpu.sync_copy(xout, xt_hbm.at[pl.ds(vbase, TV), :])

    return k(x2)


def _sc_gather(xt, idx_z, idx_y, idx_x):
    @functools.partial(
        pl.kernel,
        out_type=jax.ShapeDtypeStruct((COILS, NZ, NY, NX), jnp.float32),
        mesh=plsc.VectorSubcoreMesh(**_MESH),
        compiler_params=pltpu.CompilerParams(
            use_tc_tiling_on_sc=False, needs_layout_passes=False
        ),
        scratch_types=[
            pltpu.VMEM((ROWS, NX), jnp.int32),
            pltpu.VMEM((ROWS, NX), jnp.int32),
            pltpu.VMEM((ROWS, NX), jnp.int32),
            pltpu.VMEM((2, GR * NX, COILS), jnp.float32),
            pltpu.VMEM((COILS, GR, NX), jnp.float32),
            pltpu.SemaphoreType.DMA,
            pltpu.SemaphoreType.DMA,
        ],
    )
    def k(xt_hbm, iz_hbm, iy_hbm, ix_hbm, out_hbm,
          iz_v, iy_v, flat2, g8, crows, gsem, osem):
        wid = lax.axis_index("s") * NC + lax.axis_index("c")
        lanes = lax.iota(jnp.int32, 16)
        for chunk in range(NCHUNK):
            r0 = chunk * ROWS
            pltpu.sync_copy(iz_hbm.at[wid, pl.ds(r0, ROWS), :], iz_v)
            pltpu.sync_copy(iy_hbm.at[wid, pl.ds(r0, ROWS), :], iy_v)
            pltpu.sync_copy(ix_hbm.at[wid, pl.ds(r0, ROWS), :], flat2)

            def body(j, _):
                r = j >> 4
                s = pl.ds((j & 15) * 16, 16)
                flat2[r, s] = (
                    (iz_v[r, s] << 16) | (iy_v[r, s] << 8) | flat2[r, s]
                )
                return 0

            lax.fori_loop(0, (ROWS * NX) // 16, body, 0)

            ngroup = ROWS // GR

            def fire(g):
                def f(rr, _):
                    pltpu.async_copy(
                        xt_hbm.at[flat2.at[g * GR + rr]],
                        g8.at[g % 2, pl.ds(rr * NX, NX), :],
                        gsem,
                    )
                    return 0

                lax.fori_loop(0, GR, f, 0)

            def drain(g):
                def f(rr, _):
                    pltpu.make_async_copy(
                        xt_hbm.at[flat2.at[g * GR + rr]],
                        g8.at[g % 2, pl.ds(rr * NX, NX), :],
                        gsem,
                    ).wait()
                    return 0

                lax.fori_loop(0, GR, f, 0)

            fire(0)
            for g in range(ngroup):
                drain(g)
                if g + 1 < ngroup:
                    fire(g + 1)
                gbuf = g % 2
                for c in range(COILS):
                    cvec = jnp.full((16,), c, jnp.int32)

                    def body2(i, _, gbuf=gbuf, c=c, cvec=cvec):
                        val = plsc.load_gather(
                            g8.at[gbuf], [i * 16 + lanes, cvec]
                        )
                        crows[c, i >> 4, pl.ds((i & 15) * 16, 16)] = val
                        return 0

                    lax.fori_loop(0, (GR * NX) // 16, body2, 0)
                    pltpu.async_copy(
                        crows.at[c],
                        out_hbm.at[c, wid, pl.ds(r0 + g * GR, GR), :],
                        osem,
                    )
                for c in range(COILS):
                    pltpu.make_async_copy(
                        crows.at[c],
                        out_hbm.at[c, wid, pl.ds(r0 + g * GR, GR), :],
                        osem,
                    ).wait()

    return k(xt, idx_z, idx_y, idx_x)


def kernel(x, idx_z, idx_y, idx_x):
    xt = _transpose(x.reshape(COILS, NTOT))
    return (_sc_gather(xt, idx_z, idx_y, idx_x),)


# speedup vs baseline: 1.6129x; 1.2048x over previous
import functools

import jax
import jax.numpy as jnp
from jax import lax
from jax.experimental import pallas as pl
from jax.experimental.pallas import tpu as pltpu
from jax.experimental.pallas import tpu_sc as plsc

COILS = 8
NZ, NY, NX = 32, 256, 256
NTOT = NZ * NY * NX
NC, NS = 2, 16
NW = NC * NS

TV = 4096
TCHUNK = NTOT // NW // TV

ROWS = 32
GR = 16
NCHUNK = NY // ROWS

_MESH = dict(core_axis_name="c", subcore_axis_name="s")


def _transpose(x2):
    @functools.partial(
        pl.kernel,
        out_type=jax.ShapeDtypeStruct((NTOT, COILS), jnp.float32),
        mesh=plsc.VectorSubcoreMesh(**_MESH),
        compiler_params=pltpu.CompilerParams(
            use_tc_tiling_on_sc=False, needs_layout_passes=False
        ),
        scratch_types=[
            pltpu.VMEM((COILS, TV), jnp.float32),
            pltpu.VMEM((TV, COILS), jnp.float32),
            pltpu.SemaphoreType.DMA,
        ],
    )
    def k(x_hbm, xt_hbm, xin, xout, sem):
        wid = lax.axis_index("s") * NC + lax.axis_index("c")
        lanes = lax.iota(jnp.int32, 16)
        for chunk in range(TCHUNK):
            vbase = (wid * TCHUNK + chunk) * TV
            for c in range(COILS):
                pltpu.async_copy(
                    x_hbm.at[c, pl.ds(vbase, TV)], xin.at[c], sem
                )
            for c in range(COILS):
                pltpu.make_async_copy(
                    x_hbm.at[c, pl.ds(vbase, TV)], xin.at[c], sem
                ).wait()

            for c in range(COILS):
                cvec = jnp.full((16,), c, jnp.int32)

                @plsc.parallel_loop(0, TV, 16, unroll=8)
                def body(v0, c=c, cvec=cvec):
                    val = xin[c, pl.ds(v0, 16)]
                    plsc.store_scatter(xout, [v0 + lanes, cvec], val)

            pltpu.sync_copy(xout, xt_hbm.at[pl.ds(vbase, TV), :])

    return k(x2)


def _sc_gather(xt, idx_z, idx_y, idx_x):
    @functools.partial(
        pl.kernel,
        out_type=jax.ShapeDtypeStruct((COILS, NZ, NY, NX), jnp.float32),
        mesh=plsc.VectorSubcoreMesh(**_MESH),
        compiler_params=pltpu.CompilerParams(
            use_tc_tiling_on_sc=False, needs_layout_passes=False
        ),
        scratch_types=[
            pltpu.VMEM((ROWS, NX), jnp.int32),
            pltpu.VMEM((ROWS, NX), jnp.int32),
            pltpu.VMEM((ROWS, NX), jnp.int32),
            pltpu.VMEM((2, GR * NX, COILS), jnp.float32),
            pltpu.VMEM((COILS, GR, NX), jnp.float32),
            pltpu.SemaphoreType.DMA,
            pltpu.SemaphoreType.DMA,
        ],
    )
    def k(xt_hbm, iz_hbm, iy_hbm, ix_hbm, out_hbm,
          iz_v, iy_v, flat2, g8, crows, gsem, osem):
        wid = lax.axis_index("s") * NC + lax.axis_index("c")
        lanes = lax.iota(jnp.int32, 16)
        for chunk in range(NCHUNK):
            r0 = chunk * ROWS
            pltpu.sync_copy(iz_hbm.at[wid, pl.ds(r0, ROWS), :], iz_v)
            pltpu.sync_copy(iy_hbm.at[wid, pl.ds(r0, ROWS), :], iy_v)
            pltpu.sync_copy(ix_hbm.at[wid, pl.ds(r0, ROWS), :], flat2)

            @plsc.parallel_loop(0, (ROWS * NX) // 16, unroll=8)
            def body(j):
                r = j >> 4
                s = pl.ds((j & 15) * 16, 16)
                flat2[r, s] = (
                    (iz_v[r, s] << 16) | (iy_v[r, s] << 8) | flat2[r, s]
                )

            ngroup = ROWS // GR

            def fire(g):
                def f(rr, _):
                    pltpu.async_copy(
                        xt_hbm.at[flat2.at[g * GR + rr]],
                        g8.at[g % 2, pl.ds(rr * NX, NX), :],
                        gsem,
                    )
                    return 0

                lax.fori_loop(0, GR, f, 0)

            def drain(g):
                def f(rr, _):
                    pltpu.make_async_copy(
                        xt_hbm.at[flat2.at[g * GR + rr]],
                        g8.at[g % 2, pl.ds(rr * NX, NX), :],
                        gsem,
                    ).wait()
                    return 0

                lax.fori_loop(0, GR, f, 0)

            fire(0)
            for g in range(ngroup):
                drain(g)
                if g + 1 < ngroup:
                    fire(g + 1)
                gbuf = g % 2
                for c in range(COILS):
                    cvec = jnp.full((16,), c, jnp.int32)

                    @plsc.parallel_loop(0, (GR * NX) // 16, unroll=8)
                    def body2(i, gbuf=gbuf, c=c, cvec=cvec):
                        val = plsc.load_gather(
                            g8.at[gbuf], [i * 16 + lanes, cvec]
                        )
                        crows[c, i >> 4, pl.ds((i & 15) * 16, 16)] = val
                    pltpu.async_copy(
                        crows.at[c],
                        out_hbm.at[c, wid, pl.ds(r0 + g * GR, GR), :],
                        osem,
                    )
                for c in range(COILS):
                    pltpu.make_async_copy(
                        crows.at[c],
                        out_hbm.at[c, wid, pl.ds(r0 + g * GR, GR), :],
                        osem,
                    ).wait()

    return k(xt, idx_z, idx_y, idx_x)


def kernel(x, idx_z, idx_y, idx_x):
    xt = _transpose(x.reshape(COILS, NTOT))
    return (_sc_gather(xt, idx_z, idx_y, idx_x),)


# speedup vs baseline: 1.9881x; 1.2326x over previous
import functools

import jax
import jax.numpy as jnp
from jax import lax
from jax.experimental import pallas as pl
from jax.experimental.pallas import tpu as pltpu
from jax.experimental.pallas import tpu_sc as plsc

COILS = 8
NZ, NY, NX = 32, 256, 256
NTOT = NZ * NY * NX
NC, NS = 2, 16
NW = NC * NS

TV = 2048
TCHUNK = NTOT // NW // TV

ROWS = 32
GR = 16
NCHUNK = NY // ROWS

_MESH = dict(core_axis_name="c", subcore_axis_name="s")
_SC_PARAMS = dict(use_tc_tiling_on_sc=False, needs_layout_passes=False)


def _flat_index(idx_z, idx_y, idx_x):
    def body(iz_ref, iy_ref, ix_ref, o_ref):
        o_ref[...] = (
            (iz_ref[...] << 16) | (iy_ref[...] << 8) | ix_ref[...]
        )

    spec = pl.BlockSpec((1, NY, NX), lambda i: (i, 0, 0))
    return pl.pallas_call(
        body,
        out_shape=jax.ShapeDtypeStruct((NZ, NY, NX), jnp.int32),
        grid=(NZ,),
        in_specs=[spec, spec, spec],
        out_specs=spec,
    )(idx_z, idx_y, idx_x)


def _transpose(x2):
    @functools.partial(
        pl.kernel,
        out_type=jax.ShapeDtypeStruct((NTOT, COILS), jnp.float32),
        mesh=plsc.VectorSubcoreMesh(**_MESH),
        compiler_params=pltpu.CompilerParams(**_SC_PARAMS),
        scratch_types=[
            pltpu.VMEM((2, COILS, TV), jnp.float32),
            pltpu.VMEM((2, TV, COILS), jnp.float32),
            pltpu.SemaphoreType.DMA,
            pltpu.SemaphoreType.DMA,
        ],
    )
    def k(x_hbm, xt_hbm, xin, xout, isem, osem):
        wid = lax.axis_index("s") * NC + lax.axis_index("c")
        lanes = lax.iota(jnp.int32, 16)
        w0 = wid * TCHUNK * TV

        def in_cp(chunk, b):
            return pltpu.make_async_copy(
                x_hbm.at[:, pl.ds(w0 + chunk * TV, TV)], xin.at[b], isem
            )

        def out_cp(chunk, b):
            return pltpu.make_async_copy(
                xout.at[b], xt_hbm.at[pl.ds(w0 + chunk * TV, TV), :], osem
            )

        in_cp(0, 0).start()

        def tbody(chunk, _):
            b = chunk & 1
            in_cp(chunk, b).wait()

            @pl.when(chunk + 1 < TCHUNK)
            def _():
                in_cp(chunk + 1, 1 - b).start()

            @pl.when(chunk >= 2)
            def _():
                out_cp(chunk - 2, b).wait()

            for c in range(COILS):
                cvec = jnp.full((16,), c, jnp.int32)

                @plsc.parallel_loop(0, TV, 16, unroll=8)
                def body(v0, b=b, c=c, cvec=cvec):
                    val = xin[b, c, pl.ds(v0, 16)]
                    plsc.store_scatter(xout.at[b], [v0 + lanes, cvec], val)

            out_cp(chunk, b).start()
            return 0

        lax.fori_loop(0, TCHUNK, tbody, 0)
        out_cp(TCHUNK - 2, (TCHUNK - 2) % 2).wait()
        out_cp(TCHUNK - 1, (TCHUNK - 1) % 2).wait()

    return k(x2)


def _sc_gather(xt, flat3):
    @functools.partial(
        pl.kernel,
        out_type=jax.ShapeDtypeStruct((COILS, NZ, NY, NX), jnp.float32),
        mesh=plsc.VectorSubcoreMesh(**_MESH),
        compiler_params=pltpu.CompilerParams(**_SC_PARAMS),
        scratch_types=[
            pltpu.VMEM((2, ROWS, NX), jnp.int32),
            pltpu.VMEM((2, GR * NX, COILS), jnp.float32),
            pltpu.VMEM((COILS, GR, NX), jnp.float32),
            pltpu.SemaphoreType.DMA,
            pltpu.SemaphoreType.DMA,
            pltpu.SemaphoreType.DMA,
        ],
    )
    def k(xt_hbm, flat_hbm, out_hbm, flat2, g8, crows, fsem, gsem, osem):
        wid = lax.axis_index("s") * NC + lax.axis_index("c")
        lanes = lax.iota(jnp.int32, 16)

        def flat_cp(chunk, b):
            return pltpu.make_async_copy(
                flat_hbm.at[wid, pl.ds(chunk * ROWS, ROWS), :],
                flat2.at[b],
                fsem,
            )

        flat_cp(0, 0).start()

        def cbody(chunk, _):
            fb = chunk & 1
            r0 = chunk * ROWS
            flat_cp(chunk, fb).wait()

            @pl.when(chunk + 1 < NCHUNK)
            def _():
                flat_cp(chunk + 1, 1 - fb).start()

            ngroup = ROWS // GR

            def fire(g, fb=fb):
                def f(rr, _):
                    pltpu.async_copy(
                        xt_hbm.at[flat2.at[fb, g * GR + rr]],
                        g8.at[g % 2, pl.ds(rr * NX, NX), :],
                        gsem,
                    )
                    return 0

                lax.fori_loop(0, GR, f, 0)

            def drain(g, fb=fb):
                def f(rr, _):
                    pltpu.make_async_copy(
                        xt_hbm.at[flat2.at[fb, g * GR + rr]],
                        g8.at[g % 2, pl.ds(rr * NX, NX), :],
                        gsem,
                    ).wait()
                    return 0

                lax.fori_loop(0, GR, f, 0)

            fire(0)
            for g in range(ngroup):
                drain(g)
                if g + 1 < ngroup:
                    fire(g + 1)
                gbuf = g % 2
                for c in range(COILS):
                    cvec = jnp.full((16,), c, jnp.int32)

                    @plsc.parallel_loop(0, (GR * NX) // 16, unroll=8)
                    def body2(i, gbuf=gbuf, c=c, cvec=cvec):
                        val = plsc.load_gather(
                            g8.at[gbuf], [i * 16 + lanes, cvec]
                        )
                        crows[c, i >> 4, pl.ds((i & 15) * 16, 16)] = val

                    pltpu.async_copy(
                        crows.at[c],
                        out_hbm.at[c, wid, pl.ds(r0 + g * GR, GR), :],
                        osem,
                    )
                for c in range(COILS):
                    pltpu.make_async_copy(
                        crows.at[c],
                        out_hbm.at[c, wid, pl.ds(r0 + g * GR, GR), :],
                        osem,
                    ).wait()
            return 0

        lax.fori_loop(0, NCHUNK, cbody, 0)

    return k(xt, flat3)


def kernel(x, idx_z, idx_y, idx_x):
    flat3 = _flat_index(idx_z, idx_y, idx_x)
    xt = _transpose(x.reshape(COILS, NTOT))
    return (_sc_gather(xt, flat3),)
